# baseline (speedup 1.0000x reference)
import functools

import jax
import jax.numpy as jnp
from jax import lax
from jax.experimental import pallas as pl
from jax.experimental.pallas import tpu as pltpu
from jax.experimental.pallas import tpu_sc as plsc

N = 200000
CHUNK = 1024
NCH = 196
NPAD = NCH * CHUNK
S = 262144
CAPL = 65
FIFO = 16 * CAPL
BLK = 6
ACC = BLK * CHUNK

_TASKS = [
    (32, (0, 1, 2), 32768, 1),
    (64, (0, 1, 2), 262144, 1),
    (128, (0, 1, 2), 2097152, 8),
    (128, (0, 1), 16384, 1),
    (256, (0, 1), 65536, 1),
    (512, (0, 1), 262144, 1),
    (128, (0, 2), 16384, 1),
    (256, (0, 2), 65536, 1),
    (512, (0, 2), 262144, 1),
    (128, (1, 2), 16384, 1),
    (256, (1, 2), 65536, 1),
    (512, (1, 2), 262144, 1),
]
_CORE_TASKS = [[0, 1, 2, 3], [4, 5, 6, 7, 8, 9, 10, 11]]
_KORDER = _CORE_TASKS[0] + _CORE_TASKS[1]

_mesh = plsc.VectorSubcoreMesh(
    core_axis_name="c", subcore_axis_name="s", num_cores=2, num_subcores=16
)
_cparams = pltpu.CompilerParams(needs_layout_passes=False)
_IOTA = lambda: lax.iota(jnp.int32, 16)


def _dg(x, idx):
    return lax.gather(
        x,
        idx[:, None],
        lax.GatherDimensionNumbers(
            offset_dims=(), collapsed_slice_dims=(0,), start_index_map=(0,)
        ),
        slice_sizes=(1,),
        mode=lax.GatherScatterMode.PROMISE_IN_BOUNDS,
    )


def _vmax_scalar(v):
    for k in (1, 2, 4, 8):
        v = jnp.maximum(v, _dg(v, _IOTA() ^ k))
    return v[0]


def _corners(ndim):
    if ndim == 3:
        return [(dx, dy, dz) for dx in (0, 1) for dy in (0, 1) for dz in (0, 1)]
    return [(da, db) for da in (0, 1) for db in (0, 1)]


def _cell_frac(x, R):
    pos = jnp.clip(x, 0.0, 1.0) * float(R - 1)
    cell = jnp.minimum(pos.astype(jnp.int32), R - 2)
    frac = pos - cell.astype(jnp.float32)
    return cell, frac


def _wsel(fr, om, bit):
    return fr if bit else om


def _nchunks(sid):
    return jnp.where(sid < NCH - 16 * (NCH // 16), NCH // 16 + 1, NCH // 16)


def _zero_ref(ref, nwords):
    z = jnp.zeros((16,), jnp.float32)

    def b(i, _):
        ref[pl.ds(i * 16, 16)] = z
        return 0

    lax.fori_loop(0, nwords // 16, b, 0)


def _zero_ref_i(ref, nwords):
    z = jnp.zeros((16,), jnp.int32)

    def b(i, _):
        ref[pl.ds(i * 16, 16)] = z
        return 0

    lax.fori_loop(0, nwords // 16, b, 0)


def _emit_zero_spg(spg, zbuf, zsem, sid, cells):
    share = cells // 16
    npieces = max(1, share // 1024)
    piece = share // npieces

    def zb(i, _):
        ch = i // npieces
        p = i % npieces
        pltpu.sync_copy(
            zbuf.at[pl.ds(0, piece)],
            spg.at[pl.ds(ch * S + sid * share + p * piece, piece)],
        )
        return 0

    lax.fori_loop(0, 5 * npieces, zb, 0)


def _stage_chunk(srcs, dsts, cid_off):
    for s, d in zip(srcs, dsts):
        pltpu.sync_copy(s.at[pl.ds(cid_off, CHUNK)], d)


def _scatter_task_direct(task, coords, feats, grid, spg, stgs, idxb, valb, zsem, sid):
    R, dims, cells, nslab = task
    ndim = len(dims)
    corners = _corners(ndim)
    ncor = len(corners)
    gpf = 128 // ncor
    epg = ncor * 80
    iota = _IOTA()
    ncht = _nchunks(sid)

    def chunk_body(j, _):
        chunk = sid + j * 16
        off = chunk * CHUNK
        for d in range(ndim):
            pltpu.sync_copy(coords[dims[d]].at[pl.ds(off, CHUNK)], stgs[d])
        for q in range(4):
            pltpu.sync_copy(feats[q].at[pl.ds(off, CHUNK)], stgs[3 + q])

        def group_body(g, _):
            b = g * 16
            sg = g % gpf
            sb = sg * epg
            cellv, frv, omv = [], [], []
            for d in range(ndim):
                c, f = _cell_frac(stgs[d][pl.ds(b, 16)], R)
                cellv.append(c)
                frv.append(f)
                omv.append(1.0 - f)
            fv = [stgs[3 + q][pl.ds(b, 16)] for q in range(4)]
            pid = off + b + iota
            valid = pid < N
            frv[0] = jnp.where(valid, frv[0], 0.0)
            omv[0] = jnp.where(valid, omv[0], 0.0)
            if ndim == 3:
                base = (cellv[0] * R + cellv[1]) * R + cellv[2]
                wab = [
                    _wsel(frv[0], omv[0], da) * _wsel(frv[1], omv[1], db)
                    for da in (0, 1)
                    for db in (0, 1)
                ]
            else:
                base = cellv[0] * R + cellv[1]
                wab = None
            for ci, cc in enumerate(corners):
                if ndim == 3:
                    coff = cc[0] * R * R + cc[1] * R + cc[2]
                    w = wab[cc[0] * 2 + cc[1]] * _wsel(frv[2], omv[2], cc[2])
                else:
                    coff = cc[0] * R + cc[1]
                    w = _wsel(frv[0], omv[0], cc[0]) * _wsel(frv[1], omv[1], cc[1])
                idx = base + coff
                eb = sb + ci * 80
                for ch in range(4):
                    idxb[pl.ds(eb + ch * 16, 16)] = idx + ch * S
                    valb[pl.ds(eb + ch * 16, 16)] = w * fv[ch]
                idxb[pl.ds(eb + 64, 16)] = idx + 4 * S
                valb[pl.ds(eb + 64, 16)] = w

            @pl.when(sg == gpf - 1)
            def _fire():
                pltpu.sync_copy(valb, spg.at[idxb], add=True)

            return 0

        lax.fori_loop(0, 64, group_body, 0)
        return 0

    lax.fori_loop(0, ncht, chunk_body, 0)


def _scatter_task_slab(task, coords, feats, grid, spg, stgs, idxb, valb, fifo, zsem, sid, slab):
    R, dims, cells, nslab = task
    corners = _corners(3)
    iota = _IOTA()
    ncht = _nchunks(sid)
    x0 = slab * 16

    def chunk_body(j, _):
        chunk = sid + j * 16
        off = chunk * CHUNK
        for d in range(3):
            pltpu.sync_copy(coords[dims[d]].at[pl.ds(off, CHUNK)], stgs[d])
        for q in range(4):
            pltpu.sync_copy(feats[q].at[pl.ds(off, CHUNK)], stgs[3 + q])

        def group_body(g, cnt):
            b = g * 16
            xv = stgs[0][pl.ds(b, 16)]
            cx, _ = _cell_frac(xv, R)
            pid = off + b + iota
            act = (cx + 1 >= x0) & (cx <= x0 + 15) & (pid < N)
            pos = iota * CAPL + jnp.minimum(cnt, CAPL - 1)
            for d in range(7):
                plsc.store_scatter(fifo[d], [pos], stgs[d][pl.ds(b, 16)])
            return cnt + jnp.where(act, 1, 0)

        cnt = lax.fori_loop(0, 64, group_body, jnp.zeros((16,), jnp.int32))
        mx = _vmax_scalar(cnt)
        jmax = (mx + 3) & ~3

        def j_body(jj, _):
            sj = jj % 4
            sb = sj * 640
            idxv = iota * CAPL + jj
            valid = jj < cnt
            raw = [plsc.load_gather(fifo[d], [idxv]) for d in range(7)]
            raw = [jnp.where(valid, r, 0.0) for r in raw]
            cellv, frv, omv = [], [], []
            for d in range(3):
                c, f = _cell_frac(raw[d], R)
                cellv.append(c)
                frv.append(f)
                omv.append(1.0 - f)
            fv = raw[3:]
            base = (cellv[0] - x0) * (R * R) + cellv[1] * R + cellv[2]
            wab = [
                _wsel(frv[0], omv[0], da) * _wsel(frv[1], omv[1], db)
                for da in (0, 1)
                for db in (0, 1)
            ]
            for ci, cc in enumerate(corners):
                coff = cc[0] * R * R + cc[1] * R + cc[2]
                idx = base + coff
                inslab = (idx >= 0) & (idx < S)
                idxc = jnp.clip(idx, 0, S - 1)
                w = wab[cc[0] * 2 + cc[1]] * _wsel(frv[2], omv[2], cc[2])
                w = jnp.where(inslab & valid, w, 0.0)
                eb = sb + ci * 80
                for ch in range(4):
                    idxb[pl.ds(eb + ch * 16, 16)] = idxc + ch * S
                    valb[pl.ds(eb + ch * 16, 16)] = w * fv[ch]
                idxb[pl.ds(eb + 64, 16)] = idxc + 4 * S
                valb[pl.ds(eb + 64, 16)] = w

            @pl.when(sj == 3)
            def _fire():
                pltpu.sync_copy(valb, spg.at[idxb], add=True)

            return 0

        lax.fori_loop(0, jmax, j_body, 0)
        return 0

    lax.fori_loop(0, ncht, chunk_body, 0)


def _normalize_task(task, grid, spg, normb, zsem, sid, slab, slab_cells):
    share = slab_cells // 16
    npieces = max(1, share // 1024)
    piece = share // npieces
    goff = slab * S

    def piece_body(p, _):
        off = sid * share + p * piece
        for ch in range(5):
            pltpu.sync_copy(
                spg.at[pl.ds(ch * S + off, piece)], normb.at[pl.ds(ch * 1024, piece)]
            )

        def grp(g, _):
            b = g * 16
            cv = normb[pl.ds(4 * 1024 + b, 16)]
            r = 1.0 / jnp.maximum(cv, 1e-8)
            for ch in range(4):
                normb[pl.ds(ch * 1024 + b, 16)] = normb[pl.ds(ch * 1024 + b, 16)] * r
            return 0

        lax.fori_loop(0, piece // 16, grp, 0)
        for ch in range(4):
            pltpu.sync_copy(
                normb.at[pl.ds(ch * 1024, piece)], grid.at[ch, pl.ds(goff + off, piece)]
            )
        return 0

    lax.fori_loop(0, npieces, piece_body, 0)


def _scatter_body(cx, cy, cz, f0, f1, f2, f3, *rest):
    grids = rest[:12]
    spg, zbuf, normb, idxb, valb, idxb2, valb2 = rest[12:19]
    stgs = rest[19:26]
    fifo = rest[26:33]
    zsem = rest[33]
    cid = lax.axis_index("c")
    sid = lax.axis_index("s")
    coords = (cx, cy, cz)
    feats = (f0, f1, f2, f3)
    _zero_ref(zbuf, 1024)

    for core in (0, 1):

        @pl.when(cid == core)
        def _core_work():
            for ki, ti in enumerate(_CORE_TASKS[core]):
                task = _TASKS[ti]
                R, dims, cells, nslab = task
                grid = grids[_KORDER.index(ti)]
                slab_cells = cells // nslab
                if nslab == 1:
                    plsc.subcore_barrier()
                    _emit_zero_spg(spg, zbuf, zsem, sid, cells)
                    plsc.subcore_barrier()
                    _scatter_task_direct(
                        task, coords, feats, grid, spg, stgs, idxb, valb, zsem, sid
                    )
                    plsc.subcore_barrier()
                    _normalize_task(task, grid, spg, normb, zsem, sid, 0, cells)
                else:

                    def slab_body(slab, _):
                        plsc.subcore_barrier()
                        _emit_zero_spg(spg, zbuf, zsem, sid, slab_cells)
                        plsc.subcore_barrier()
                        _scatter_task_slab(
                            task, coords, feats, grid, spg, stgs, idxb2, valb2,
                            fifo, zsem, sid, slab,
                        )
                        plsc.subcore_barrier()
                        _normalize_task(task, grid, spg, normb, zsem, sid, slab, slab_cells)
                        return 0

                    lax.fori_loop(0, nslab, slab_body, 0)


_scatter_kernel = functools.partial(
    pl.kernel,
    out_type=tuple(
        jax.ShapeDtypeStruct((4, _TASKS[t][2]), jnp.float32) for t in _KORDER
    ),
    mesh=_mesh,
    compiler_params=_cparams,
    scratch_types=(
        [pltpu.VMEM_SHARED((5 * S,), jnp.float32)]
        + [pltpu.VMEM((1024,), jnp.float32)]
        + [pltpu.VMEM((5120,), jnp.float32)]
        + [pltpu.VMEM((10240,), jnp.int32), pltpu.VMEM((10240,), jnp.float32)]
        + [pltpu.VMEM((2560,), jnp.int32), pltpu.VMEM((2560,), jnp.float32)]
        + [pltpu.VMEM((CHUNK,), jnp.float32) for _ in range(7)]
        + [pltpu.VMEM((FIFO,), jnp.float32) for _ in range(7)]
        + [pltpu.SemaphoreType.DMA]
    ),
)(_scatter_body)


def _gather_task_direct(task, coords, ctx, spg, stgs, idxb, dstb, wb, outb, zsem, sid):
    R, dims, cells, nslab = task
    ndim = len(dims)
    corners = _corners(ndim)
    ncor = len(corners)
    gpf = 16 if ndim == 3 else 32
    epg = ncor * 64
    iota = _IOTA()
    ncht = _nchunks(sid)

    def chunk_body(j, _):
        chunk = sid + j * 16
        off = chunk * CHUNK
        for d in range(ndim):
            pltpu.sync_copy(coords[dims[d]].at[pl.ds(off, CHUNK)], stgs[d])

        def group_body(g, _):
            b = g * 16
            sg = g % gpf
            sb = sg * epg
            cellv, frv, omv = [], [], []
            for d in range(ndim):
                c, f = _cell_frac(stgs[d][pl.ds(b, 16)], R)
                cellv.append(c)
                frv.append(f)
                omv.append(1.0 - f)
            if ndim == 3:
                base = (cellv[0] * R + cellv[1]) * R + cellv[2]
                wab = [
                    _wsel(frv[0], omv[0], da) * _wsel(frv[1], omv[1], db)
                    for da in (0, 1)
                    for db in (0, 1)
                ]
            else:
                base = cellv[0] * R + cellv[1]
            for ci, cc in enumerate(corners):
                if ndim == 3:
                    coff = cc[0] * R * R + cc[1] * R + cc[2]
                    w = wab[cc[0] * 2 + cc[1]] * _wsel(frv[2], omv[2], cc[2])
                else:
                    coff = cc[0] * R + cc[1]
                    w = _wsel(frv[0], omv[0], cc[0]) * _wsel(frv[1], omv[1], cc[1])
                idx = base + coff
                eb = sb + ci * 64
                for ch in range(4):
                    idxb[pl.ds(eb + ch * 16, 16)] = idx + ch * S
                wb[pl.ds(sg * ncor * 16 + ci * 16, 16)] = w

            @pl.when(sg == gpf - 1)
            def _fire_combine():
                pltpu.sync_copy(spg.at[idxb], dstb)

                def comb(k, _):
                    g2 = g - (gpf - 1) + k
                    b2 = g2 * 16
                    sb2 = k * epg
                    acc = [jnp.zeros((16,), jnp.float32) for _ in range(4)]
                    for ci in range(ncor):
                        w = wb[pl.ds(k * ncor * 16 + ci * 16, 16)]
                        for ch in range(4):
                            acc[ch] = acc[ch] + w * dstb[pl.ds(sb2 + ci * 64 + ch * 16, 16)]
                    for ch in range(4):
                        outb[pl.ds(ch * CHUNK + b2, 16)] = acc[ch]
                    return 0

                lax.fori_loop(0, gpf, comb, 0)

            return 0

        lax.fori_loop(0, 64, group_body, 0)
        for ch in range(4):
            pltpu.sync_copy(
                outb.at[pl.ds(ch * CHUNK, CHUNK)], ctx.at[ch, pl.ds(off, CHUNK)]
            )
        return 0

    lax.fori_loop(0, ncht, chunk_body, 0)


def _gather_body(ix, iy, iz, *rest):
    grids = rest[:12]
    ctxs = rest[12:24]
    spg = rest[24]
    stgs = rest[25:28]
    idxb, dstb, idxb2, dstb2, wb, outb, idsb = rest[28:35]
    gfifo = rest[35:39]
    gacc = rest[39:43]
    zsem = rest[43]
    cid = lax.axis_index("c")
    sid = lax.axis_index("s")
    iota = _IOTA()
    coords = (ix, iy, iz)
    corners = _corners(3)

    for core in (0, 1):

        @pl.when(cid == core)
        def _core_work():
            for ki, ti in enumerate(_CORE_TASKS[core]):
                task = _TASKS[ti]
                R, dims, cells, nslab = task
                ko = _KORDER.index(ti)
                grid = grids[ko]
                ctx = ctxs[ko]
                slab_cells = cells // nslab
                share = slab_cells // 16
                if nslab == 1:
                    plsc.subcore_barrier()
                    for ch in range(4):
                        pltpu.sync_copy(
                            grid.at[ch, pl.ds(sid * share, share)],
                            spg.at[pl.ds(ch * S + sid * share, share)],
                        )
                    plsc.subcore_barrier()
                    _gather_task_direct(
                        task, coords, ctx, spg, stgs, idxb, dstb, wb, outb, zsem, sid
                    )
                else:
                    ncht = _nchunks(sid)
                    for blk in range(3):
                        bcnt = jnp.minimum(ncht - blk * BLK, BLK)
                        for ch in range(4):
                            _zero_ref(gacc[ch], ACC)

                        def slab_body(slab, _):
                            x0 = slab * 16
                            plsc.subcore_barrier()
                            for ch in range(4):
                                pltpu.sync_copy(
                                    grid.at[ch, pl.ds(slab * S + sid * share, share)],
                                    spg.at[pl.ds(ch * S + sid * share, share)],
                                )
                            plsc.subcore_barrier()

                            def chunk_body(jc, _):
                                chunk = sid + (blk * BLK + jc) * 16
                                off = chunk * CHUNK
                                for d in range(3):
                                    pltpu.sync_copy(
                                        coords[dims[d]].at[pl.ds(off, CHUNK)], stgs[d]
                                    )

                                def group_body(g, cnt):
                                    b = g * 16
                                    xv = stgs[0][pl.ds(b, 16)]
                                    cx, _ = _cell_frac(xv, R)
                                    act = (cx + 1 >= x0) & (cx <= x0 + 15)
                                    pos = iota * CAPL + jnp.minimum(cnt, CAPL - 1)
                                    for d in range(3):
                                        plsc.store_scatter(
                                            gfifo[d], [pos], stgs[d][pl.ds(b, 16)]
                                        )
                                    plsc.store_scatter(
                                        gfifo[3], [pos], jc * CHUNK + b + iota
                                    )
                                    return cnt + jnp.where(act, 1, 0)

                                cnt = lax.fori_loop(
                                    0, 64, group_body, jnp.zeros((16,), jnp.int32)
                                )
                                mx = _vmax_scalar(cnt)
                                jmax = (mx + 7) & ~7

                                def j_body(j, _):
                                    sj = j % 8
                                    sb = sj * 512
                                    idxv = iota * CAPL + j
                                    valid = j < cnt
                                    raw = [
                                        plsc.load_gather(gfifo[d], [idxv])
                                        for d in range(3)
                                    ]
                                    raw = [jnp.where(valid, r, 0.0) for r in raw]
                                    ids = plsc.load_gather(gfifo[3], [idxv])
                                    ids = jnp.where(valid, ids, 0)
                                    ids = jnp.clip(ids, 0, ACC - 1)
                                    idsb[pl.ds(sj * 16, 16)] = ids
                                    cellv, frv, omv = [], [], []
                                    for d in range(3):
                                        c, f = _cell_frac(raw[d], R)
                                        cellv.append(c)
                                        frv.append(f)
                                        omv.append(1.0 - f)
                                    base = (
                                        (cellv[0] - x0) * (R * R)
                                        + cellv[1] * R
                                        + cellv[2]
                                    )
                                    wab = [
                                        _wsel(frv[0], omv[0], da)
                                        * _wsel(frv[1], omv[1], db)
                                        for da in (0, 1)
                                        for db in (0, 1)
                                    ]
                                    for ci, cc in enumerate(corners):
                                        coff = cc[0] * R * R + cc[1] * R + cc[2]
                                        idx = base + coff
                                        inslab = (idx >= 0) & (idx < S)
                                        idxc = jnp.clip(idx, 0, S - 1)
                                        w = wab[cc[0] * 2 + cc[1]] * _wsel(
                                            frv[2], omv[2], cc[2]
                                        )
                                        w = jnp.where(inslab & valid, w, 0.0)
                                        eb = sb + ci * 64
                                        for ch in range(4):
                                            idxb2[pl.ds(eb + ch * 16, 16)] = (
                                                idxc + ch * S
                                            )
                                        wb[pl.ds(sj * 8 * 16 + ci * 16, 16)] = w

                                    @pl.when(sj == 7)
                                    def _fire_combine():
                                        pltpu.sync_copy(spg.at[idxb2], dstb2)

                                        def comb(k, _):
                                            sb2 = k * 512
                                            ids2 = idsb[pl.ds(k * 16, 16)]
                                            acc = [
                                                jnp.zeros((16,), jnp.float32)
                                                for _ in range(4)
                                            ]
                                            for ci in range(8):
                                                w = wb[pl.ds(k * 8 * 16 + ci * 16, 16)]
                                                for ch in range(4):
                                                    acc[ch] = acc[ch] + w * dstb2[
                                                        pl.ds(sb2 + ci * 64 + ch * 16, 16)
                                                    ]
                                            for ch in range(4):
                                                plsc.addupdate_scatter(
                                                    gacc[ch], [ids2], acc[ch]
                                                )
                                            return 0

                                        lax.fori_loop(0, 8, comb, 0)

                                    return 0

                                lax.fori_loop(0, jmax, j_body, 0)
                                return 0

                            lax.fori_loop(0, bcnt, chunk_body, 0)
                            return 0

                        lax.fori_loop(0, nslab, slab_body, 0)

                        def wb_body(jc, _):
                            chunk = sid + (blk * BLK + jc) * 16
                            for ch in range(4):
                                pltpu.sync_copy(
                                    gacc[ch].at[pl.ds(jc * CHUNK, CHUNK)],
                                    ctx.at[ch, pl.ds(chunk * CHUNK, CHUNK)],
                                )
                            return 0

                        lax.fori_loop(0, bcnt, wb_body, 0)


_gather_kernel = functools.partial(
    pl.kernel,
    out_type=tuple(
        jax.ShapeDtypeStruct((4, NPAD), jnp.float32) for _ in range(12)
    ),
    mesh=_mesh,
    compiler_params=_cparams,
    scratch_types=(
        [pltpu.VMEM_SHARED((4 * S,), jnp.float32)]
        + [pltpu.VMEM((CHUNK,), jnp.float32) for _ in range(3)]
        + [
            pltpu.VMEM((8192,), jnp.int32),
            pltpu.VMEM((8192,), jnp.float32),
            pltpu.VMEM((4096,), jnp.int32),
            pltpu.VMEM((4096,), jnp.float32),
            pltpu.VMEM((2048,), jnp.float32),
            pltpu.VMEM((4 * CHUNK,), jnp.float32),
            pltpu.VMEM((128,), jnp.int32),
        ]
        + [pltpu.VMEM((FIFO,), jnp.float32) for _ in range(3)]
        + [pltpu.VMEM((FIFO,), jnp.int32)]
        + [pltpu.VMEM((ACC,), jnp.float32) for _ in range(4)]
        + [pltpu.SemaphoreType.DMA]
    ),
)(_gather_body)


def kernel(xyz_for_creater, xyz_for_interp, feature):
    pad = NPAD - N
    cx = jnp.pad(xyz_for_creater[:, 0], (0, pad))
    cy = jnp.pad(xyz_for_creater[:, 1], (0, pad))
    cz = jnp.pad(xyz_for_creater[:, 2], (0, pad))
    f0 = jnp.pad(feature[:, 0], (0, pad))
    f1 = jnp.pad(feature[:, 1], (0, pad))
    f2 = jnp.pad(feature[:, 2], (0, pad))
    f3 = jnp.pad(feature[:, 3], (0, pad))
    ix = jnp.pad(xyz_for_interp[:, 0], (0, pad))
    iy = jnp.pad(xyz_for_interp[:, 1], (0, pad))
    iz = jnp.pad(xyz_for_interp[:, 2], (0, pad))
    grids = _scatter_kernel(cx, cy, cz, f0, f1, f2, f3)
    ctxs = _gather_kernel(ix, iy, iz, *grids)
    by_task = {ti: ctxs[k] for k, ti in enumerate(_KORDER)}
    cols = [by_task[t][:, :N].T for t in range(12)]
    return jnp.concatenate(cols, axis=1)


# speedup vs baseline: 1.0701x; 1.0701x over previous
import functools

import jax
import jax.numpy as jnp
from jax import lax
from jax.experimental import pallas as pl
from jax.experimental.pallas import tpu as pltpu
from jax.experimental.pallas import tpu_sc as plsc

N = 200000
CHUNK = 1024
NCH = 196
NPAD = NCH * CHUNK
S = 262144
CAPL = 65
FIFO = 16 * CAPL
BLK = 8
ACC = BLK * CHUNK

_TASKS = [
    (32, (0, 1, 2), 32768, 1),
    (64, (0, 1, 2), 262144, 1),
    (128, (0, 1, 2), 2097152, 8),
    (128, (0, 1), 16384, 1),
    (256, (0, 1), 65536, 1),
    (512, (0, 1), 262144, 1),
    (128, (0, 2), 16384, 1),
    (256, (0, 2), 65536, 1),
    (512, (0, 2), 262144, 1),
    (128, (1, 2), 16384, 1),
    (256, (1, 2), 65536, 1),
    (512, (1, 2), 262144, 1),
]
_CORE_TASKS = [[0, 1, 2, 3], [4, 5, 6, 7, 8, 9, 10, 11]]
_KORDER = _CORE_TASKS[0] + _CORE_TASKS[1]

_mesh = plsc.VectorSubcoreMesh(
    core_axis_name="c", subcore_axis_name="s", num_cores=2, num_subcores=16
)
_cparams = pltpu.CompilerParams(needs_layout_passes=False)
_IOTA = lambda: lax.iota(jnp.int32, 16)


def _dg(x, idx):
    return lax.gather(
        x,
        idx[:, None],
        lax.GatherDimensionNumbers(
            offset_dims=(), collapsed_slice_dims=(0,), start_index_map=(0,)
        ),
        slice_sizes=(1,),
        mode=lax.GatherScatterMode.PROMISE_IN_BOUNDS,
    )


def _vmax_scalar(v):
    for k in (1, 2, 4, 8):
        v = jnp.maximum(v, _dg(v, _IOTA() ^ k))
    return v[0]


def _corners(ndim):
    if ndim == 3:
        return [(dx, dy, dz) for dx in (0, 1) for dy in (0, 1) for dz in (0, 1)]
    return [(da, db) for da in (0, 1) for db in (0, 1)]


def _cell_frac(x, R):
    pos = jnp.clip(x, 0.0, 1.0) * float(R - 1)
    cell = jnp.minimum(pos.astype(jnp.int32), R - 2)
    frac = pos - cell.astype(jnp.float32)
    return cell, frac


def _wsel(fr, om, bit):
    return fr if bit else om


def _nchunks(sid):
    return jnp.where(sid < NCH - 16 * (NCH // 16), NCH // 16 + 1, NCH // 16)


def _zero_ref(ref, nwords):
    z = jnp.zeros((16,), jnp.float32)

    def b(i, _):
        ref[pl.ds(i * 16, 16)] = z
        return 0

    lax.fori_loop(0, nwords // 16, b, 0)


def _zero_ref_i(ref, nwords):
    z = jnp.zeros((16,), jnp.int32)

    def b(i, _):
        ref[pl.ds(i * 16, 16)] = z
        return 0

    lax.fori_loop(0, nwords // 16, b, 0)


def _emit_zero_spg(spg, zbuf, zsem, sid, cells):
    share = cells // 16
    npieces = max(1, share // 1024)
    piece = share // npieces

    def zb(i, _):
        ch = i // npieces
        p = i % npieces
        pltpu.sync_copy(
            zbuf.at[pl.ds(0, piece)],
            spg.at[pl.ds(ch * S + sid * share + p * piece, piece)],
        )
        return 0

    lax.fori_loop(0, 5 * npieces, zb, 0)


def _stage_chunk(srcs, dsts, cid_off):
    for s, d in zip(srcs, dsts):
        pltpu.sync_copy(s.at[pl.ds(cid_off, CHUNK)], d)


def _scatter_task_direct(task, coords, feats, grid, spg, stgs, idxb, valb, zsem, sid):
    R, dims, cells, nslab = task
    ndim = len(dims)
    corners = _corners(ndim)
    ncor = len(corners)
    gpf = 64 // ncor
    epg = ncor * 80
    iota = _IOTA()
    ncht = _nchunks(sid)

    def chunk_body(j, _):
        chunk = sid + j * 16
        off = chunk * CHUNK
        for d in range(ndim):
            pltpu.sync_copy(coords[dims[d]].at[pl.ds(off, CHUNK)], stgs[d])
        for q in range(4):
            pltpu.sync_copy(feats[q].at[pl.ds(off, CHUNK)], stgs[3 + q])

        def group_body(g, _):
            b = g * 16
            sg = g % gpf
            sb = sg * epg
            cellv, frv, omv = [], [], []
            for d in range(ndim):
                c, f = _cell_frac(stgs[d][pl.ds(b, 16)], R)
                cellv.append(c)
                frv.append(f)
                omv.append(1.0 - f)
            fv = [stgs[3 + q][pl.ds(b, 16)] for q in range(4)]
            pid = off + b + iota
            valid = pid < N
            frv[0] = jnp.where(valid, frv[0], 0.0)
            omv[0] = jnp.where(valid, omv[0], 0.0)
            if ndim == 3:
                base = (cellv[0] * R + cellv[1]) * R + cellv[2]
                wab = [
                    _wsel(frv[0], omv[0], da) * _wsel(frv[1], omv[1], db)
                    for da in (0, 1)
                    for db in (0, 1)
                ]
            else:
                base = cellv[0] * R + cellv[1]
                wab = None
            for ci, cc in enumerate(corners):
                if ndim == 3:
                    coff = cc[0] * R * R + cc[1] * R + cc[2]
                    w = wab[cc[0] * 2 + cc[1]] * _wsel(frv[2], omv[2], cc[2])
                else:
                    coff = cc[0] * R + cc[1]
                    w = _wsel(frv[0], omv[0], cc[0]) * _wsel(frv[1], omv[1], cc[1])
                idx = base + coff
                eb = sb + ci * 80
                for ch in range(4):
                    idxb[pl.ds(eb + ch * 16, 16)] = idx + ch * S
                    valb[pl.ds(eb + ch * 16, 16)] = w * fv[ch]
                idxb[pl.ds(eb + 64, 16)] = idx + 4 * S
                valb[pl.ds(eb + 64, 16)] = w

            @pl.when(sg == gpf - 1)
            def _fire():
                pltpu.sync_copy(valb, spg.at[idxb], add=True)

            return 0

        lax.fori_loop(0, 64, group_body, 0)
        return 0

    lax.fori_loop(0, ncht, chunk_body, 0)


def _scatter_task_slab(task, coords, feats, grid, spg, stgs, idxb, valb, fifo, zsem, sid, slab):
    R, dims, cells, nslab = task
    corners = _corners(3)
    iota = _IOTA()
    ncht = _nchunks(sid)
    x0 = slab * 16

    def chunk_body(j, _):
        chunk = sid + j * 16
        off = chunk * CHUNK
        for d in range(3):
            pltpu.sync_copy(coords[dims[d]].at[pl.ds(off, CHUNK)], stgs[d])
        for q in range(4):
            pltpu.sync_copy(feats[q].at[pl.ds(off, CHUNK)], stgs[3 + q])

        def group_body(g, cnt):
            b = g * 16
            xv = stgs[0][pl.ds(b, 16)]
            cx, _ = _cell_frac(xv, R)
            pid = off + b + iota
            act = (cx + 1 >= x0) & (cx <= x0 + 15) & (pid < N)
            pos = iota * CAPL + jnp.minimum(cnt, CAPL - 1)
            for d in range(7):
                plsc.store_scatter(fifo[d], [pos], stgs[d][pl.ds(b, 16)])
            return cnt + jnp.where(act, 1, 0)

        cnt = lax.fori_loop(0, 64, group_body, jnp.zeros((16,), jnp.int32))
        mx = _vmax_scalar(cnt)
        jmax = (mx + 3) & ~3

        def j_body(jj, _):
            sj = jj % 4
            sb = sj * 640
            idxv = iota * CAPL + jj
            valid = jj < cnt
            raw = [plsc.load_gather(fifo[d], [idxv]) for d in range(7)]
            raw = [jnp.where(valid, r, 0.0) for r in raw]
            cellv, frv, omv = [], [], []
            for d in range(3):
                c, f = _cell_frac(raw[d], R)
                cellv.append(c)
                frv.append(f)
                omv.append(1.0 - f)
            fv = raw[3:]
            base = (cellv[0] - x0) * (R * R) + cellv[1] * R + cellv[2]
            wab = [
                _wsel(frv[0], omv[0], da) * _wsel(frv[1], omv[1], db)
                for da in (0, 1)
                for db in (0, 1)
            ]
            for ci, cc in enumerate(corners):
                coff = cc[0] * R * R + cc[1] * R + cc[2]
                idx = base + coff
                inslab = (idx >= 0) & (idx < S)
                idxc = jnp.clip(idx, 0, S - 1)
                w = wab[cc[0] * 2 + cc[1]] * _wsel(frv[2], omv[2], cc[2])
                w = jnp.where(inslab & valid, w, 0.0)
                eb = sb + ci * 80
                for ch in range(4):
                    idxb[pl.ds(eb + ch * 16, 16)] = idxc + ch * S
                    valb[pl.ds(eb + ch * 16, 16)] = w * fv[ch]
                idxb[pl.ds(eb + 64, 16)] = idxc + 4 * S
                valb[pl.ds(eb + 64, 16)] = w

            @pl.when(sj == 3)
            def _fire():
                pltpu.sync_copy(valb, spg.at[idxb], add=True)

            return 0

        lax.fori_loop(0, jmax, j_body, 0)
        return 0

    lax.fori_loop(0, ncht, chunk_body, 0)


def _normalize_task(task, grid, spg, normb, zsem, sid, slab, slab_cells):
    share = slab_cells // 16
    npieces = max(1, share // 1024)
    piece = share // npieces
    goff = slab * S

    def piece_body(p, _):
        off = sid * share + p * piece
        for ch in range(5):
            pltpu.sync_copy(
                spg.at[pl.ds(ch * S + off, piece)], normb.at[pl.ds(ch * 1024, piece)]
            )

        def grp(g, _):
            b = g * 16
            cv = normb[pl.ds(4 * 1024 + b, 16)]
            r = 1.0 / jnp.maximum(cv, 1e-8)
            for ch in range(4):
                normb[pl.ds(ch * 1024 + b, 16)] = normb[pl.ds(ch * 1024 + b, 16)] * r
            return 0

        lax.fori_loop(0, piece // 16, grp, 0)
        for ch in range(4):
            pltpu.sync_copy(
                normb.at[pl.ds(ch * 1024, piece)], grid.at[ch, pl.ds(goff + off, piece)]
            )
        return 0

    lax.fori_loop(0, npieces, piece_body, 0)


def _scatter_body(cx, cy, cz, f0, f1, f2, f3, *rest):
    grids = rest[:12]
    spg, zbuf, normb, idxb, valb, idxb2, valb2 = rest[12:19]
    stgs = rest[19:26]
    fifo = rest[26:33]
    zsem = rest[33]
    cid = lax.axis_index("c")
    sid = lax.axis_index("s")
    coords = (cx, cy, cz)
    feats = (f0, f1, f2, f3)
    _zero_ref(zbuf, 1024)

    for core in (0, 1):

        @pl.when(cid == core)
        def _core_work():
            for ki, ti in enumerate(_CORE_TASKS[core]):
                task = _TASKS[ti]
                R, dims, cells, nslab = task
                grid = grids[_KORDER.index(ti)]
                slab_cells = cells // nslab
                if nslab == 1:
                    plsc.subcore_barrier()
                    _emit_zero_spg(spg, zbuf, zsem, sid, cells)
                    plsc.subcore_barrier()
                    _scatter_task_direct(
                        task, coords, feats, grid, spg, stgs, idxb, valb, zsem, sid
                    )
                    plsc.subcore_barrier()
                    _normalize_task(task, grid, spg, normb, zsem, sid, 0, cells)
                else:

                    def slab_body(slab, _):
                        plsc.subcore_barrier()
                        _emit_zero_spg(spg, zbuf, zsem, sid, slab_cells)
                        plsc.subcore_barrier()
                        _scatter_task_slab(
                            task, coords, feats, grid, spg, stgs, idxb2, valb2,
                            fifo, zsem, sid, slab,
                        )
                        plsc.subcore_barrier()
                        _normalize_task(task, grid, spg, normb, zsem, sid, slab, slab_cells)
                        return 0

                    lax.fori_loop(0, nslab, slab_body, 0)


_scatter_kernel = functools.partial(
    pl.kernel,
    out_type=tuple(
        jax.ShapeDtypeStruct((4, _TASKS[t][2]), jnp.float32) for t in _KORDER
    ),
    mesh=_mesh,
    compiler_params=_cparams,
    scratch_types=(
        [pltpu.VMEM_SHARED((5 * S,), jnp.float32)]
        + [pltpu.VMEM((1024,), jnp.float32)]
        + [pltpu.VMEM((5120,), jnp.float32)]
        + [pltpu.VMEM((5120,), jnp.int32), pltpu.VMEM((5120,), jnp.float32)]
        + [pltpu.VMEM((2560,), jnp.int32), pltpu.VMEM((2560,), jnp.float32)]
        + [pltpu.VMEM((CHUNK,), jnp.float32) for _ in range(7)]
        + [pltpu.VMEM((FIFO,), jnp.float32) for _ in range(7)]
        + [pltpu.SemaphoreType.DMA]
    ),
)(_scatter_body)


def _gather_task_direct(task, coords, ctx, spg, stgs, idxb, dstb, wb, outb, zsem, sid):
    R, dims, cells, nslab = task
    ndim = len(dims)
    corners = _corners(ndim)
    ncor = len(corners)
    gpf = 8 if ndim == 3 else 16
    epg = ncor * 64
    iota = _IOTA()
    ncht = _nchunks(sid)

    def chunk_body(j, _):
        chunk = sid + j * 16
        off = chunk * CHUNK
        for d in range(ndim):
            pltpu.sync_copy(coords[dims[d]].at[pl.ds(off, CHUNK)], stgs[d])

        def group_body(g, _):
            b = g * 16
            sg = g % gpf
            sb = sg * epg
            cellv, frv, omv = [], [], []
            for d in range(ndim):
                c, f = _cell_frac(stgs[d][pl.ds(b, 16)], R)
                cellv.append(c)
                frv.append(f)
                omv.append(1.0 - f)
            if ndim == 3:
                base = (cellv[0] * R + cellv[1]) * R + cellv[2]
                wab = [
                    _wsel(frv[0], omv[0], da) * _wsel(frv[1], omv[1], db)
                    for da in (0, 1)
                    for db in (0, 1)
                ]
            else:
                base = cellv[0] * R + cellv[1]
            for ci, cc in enumerate(corners):
                if ndim == 3:
                    coff = cc[0] * R * R + cc[1] * R + cc[2]
                    w = wab[cc[0] * 2 + cc[1]] * _wsel(frv[2], omv[2], cc[2])
                else:
                    coff = cc[0] * R + cc[1]
                    w = _wsel(frv[0], omv[0], cc[0]) * _wsel(frv[1], omv[1], cc[1])
                idx = base + coff
                eb = sb + ci * 64
                for ch in range(4):
                    idxb[pl.ds(eb + ch * 16, 16)] = idx + ch * S
                wb[pl.ds(sg * ncor * 16 + ci * 16, 16)] = w

            @pl.when(sg == gpf - 1)
            def _fire_combine():
                pltpu.sync_copy(spg.at[idxb], dstb)

                def comb(k, _):
                    g2 = g - (gpf - 1) + k
                    b2 = g2 * 16
                    sb2 = k * epg
                    acc = [jnp.zeros((16,), jnp.float32) for _ in range(4)]
                    for ci in range(ncor):
                        w = wb[pl.ds(k * ncor * 16 + ci * 16, 16)]
                        for ch in range(4):
                            acc[ch] = acc[ch] + w * dstb[pl.ds(sb2 + ci * 64 + ch * 16, 16)]
                    for ch in range(4):
                        outb[pl.ds(ch * CHUNK + b2, 16)] = acc[ch]
                    return 0

                lax.fori_loop(0, gpf, comb, 0)

            return 0

        lax.fori_loop(0, 64, group_body, 0)
        for ch in range(4):
            pltpu.sync_copy(
                outb.at[pl.ds(ch * CHUNK, CHUNK)], ctx.at[ch, pl.ds(off, CHUNK)]
            )
        return 0

    lax.fori_loop(0, ncht, chunk_body, 0)


def _gather_body(ix, iy, iz, *rest):
    grids = rest[:12]
    ctxs = rest[12:24]
    spg = rest[24]
    stgs = rest[25:28]
    idxb, dstb, idxb2, dstb2, wb, outb, idsb = rest[28:35]
    gfifo = rest[35:39]
    gacc = rest[39:43]
    zsem = rest[43]
    cid = lax.axis_index("c")
    sid = lax.axis_index("s")
    iota = _IOTA()
    coords = (ix, iy, iz)
    corners = _corners(3)

    for core in (0, 1):

        @pl.when(cid == core)
        def _core_work():
            for ki, ti in enumerate(_CORE_TASKS[core]):
                task = _TASKS[ti]
                R, dims, cells, nslab = task
                ko = _KORDER.index(ti)
                grid = grids[ko]
                ctx = ctxs[ko]
                slab_cells = cells // nslab
                share = slab_cells // 16
                if nslab == 1:
                    plsc.subcore_barrier()
                    for ch in range(4):
                        pltpu.sync_copy(
                            grid.at[ch, pl.ds(sid * share, share)],
                            spg.at[pl.ds(ch * S + sid * share, share)],
                        )
                    plsc.subcore_barrier()
                    _gather_task_direct(
                        task, coords, ctx, spg, stgs, idxb, dstb, wb, outb, zsem, sid
                    )
                else:
                    ncht = _nchunks(sid)
                    for blk in range(2):
                        bcnt = jnp.minimum(ncht - blk * BLK, BLK)
                        for ch in range(4):
                            _zero_ref(gacc[ch], ACC)

                        def slab_body(slab, _):
                            x0 = slab * 16
                            plsc.subcore_barrier()
                            for ch in range(4):
                                pltpu.sync_copy(
                                    grid.at[ch, pl.ds(slab * S + sid * share, share)],
                                    spg.at[pl.ds(ch * S + sid * share, share)],
                                )
                            plsc.subcore_barrier()

                            def chunk_body(jc, _):
                                chunk = sid + (blk * BLK + jc) * 16
                                off = chunk * CHUNK
                                for d in range(3):
                                    pltpu.sync_copy(
                                        coords[dims[d]].at[pl.ds(off, CHUNK)], stgs[d]
                                    )

                                def group_body(g, cnt):
                                    b = g * 16
                                    xv = stgs[0][pl.ds(b, 16)]
                                    cx, _ = _cell_frac(xv, R)
                                    act = (cx + 1 >= x0) & (cx <= x0 + 15)
                                    pos = iota * CAPL + jnp.minimum(cnt, CAPL - 1)
                                    for d in range(3):
                                        plsc.store_scatter(
                                            gfifo[d], [pos], stgs[d][pl.ds(b, 16)]
                                        )
                                    plsc.store_scatter(
                                        gfifo[3], [pos], jc * CHUNK + b + iota
                                    )
                                    return cnt + jnp.where(act, 1, 0)

                                cnt = lax.fori_loop(
                                    0, 64, group_body, jnp.zeros((16,), jnp.int32)
                                )
                                mx = _vmax_scalar(cnt)
                                jmax = (mx + 3) & ~3

                                def j_body(j, _):
                                    sj = j % 4
                                    sb = sj * 512
                                    idxv = iota * CAPL + j
                                    valid = j < cnt
                                    raw = [
                                        plsc.load_gather(gfifo[d], [idxv])
                                        for d in range(3)
                                    ]
                                    raw = [jnp.where(valid, r, 0.0) for r in raw]
                                    ids = plsc.load_gather(gfifo[3], [idxv])
                                    ids = jnp.where(valid, ids, 0)
                                    ids = jnp.clip(ids, 0, ACC - 1)
                                    idsb[pl.ds(sj * 16, 16)] = ids
                                    cellv, frv, omv = [], [], []
                                    for d in range(3):
                                        c, f = _cell_frac(raw[d], R)
                                        cellv.append(c)
                                        frv.append(f)
                                        omv.append(1.0 - f)
                                    base = (
                                        (cellv[0] - x0) * (R * R)
                                        + cellv[1] * R
                                        + cellv[2]
                                    )
                                    wab = [
                                        _wsel(frv[0], omv[0], da)
                                        * _wsel(frv[1], omv[1], db)
                                        for da in (0, 1)
                                        for db in (0, 1)
                                    ]
                                    for ci, cc in enumerate(corners):
                                        coff = cc[0] * R * R + cc[1] * R + cc[2]
                                        idx = base + coff
                                        inslab = (idx >= 0) & (idx < S)
                                        idxc = jnp.clip(idx, 0, S - 1)
                                        w = wab[cc[0] * 2 + cc[1]] * _wsel(
                                            frv[2], omv[2], cc[2]
                                        )
                                        w = jnp.where(inslab & valid, w, 0.0)
                                        eb = sb + ci * 64
                                        for ch in range(4):
                                            idxb2[pl.ds(eb + ch * 16, 16)] = (
                                                idxc + ch * S
                                            )
                                        wb[pl.ds(sj * 8 * 16 + ci * 16, 16)] = w

                                    @pl.when(sj == 3)
                                    def _fire_combine():
                                        pltpu.sync_copy(spg.at[idxb2], dstb2)

                                        def comb(k, _):
                                            sb2 = k * 512
                                            ids2 = idsb[pl.ds(k * 16, 16)]
                                            acc = [
                                                jnp.zeros((16,), jnp.float32)
                                                for _ in range(4)
                                            ]
                                            for ci in range(8):
                                                w = wb[pl.ds(k * 8 * 16 + ci * 16, 16)]
                                                for ch in range(4):
                                                    acc[ch] = acc[ch] + w * dstb2[
                                                        pl.ds(sb2 + ci * 64 + ch * 16, 16)
                                                    ]
                                            for ch in range(4):
                                                plsc.addupdate_scatter(
                                                    gacc[ch], [ids2], acc[ch]
                                                )
                                            return 0

                                        lax.fori_loop(0, 4, comb, 0)

                                    return 0

                                lax.fori_loop(0, jmax, j_body, 0)
                                return 0

                            lax.fori_loop(0, bcnt, chunk_body, 0)
                            return 0

                        lax.fori_loop(0, nslab, slab_body, 0)

                        def wb_body(jc, _):
                            chunk = sid + (blk * BLK + jc) * 16
                            for ch in range(4):
                                pltpu.sync_copy(
                                    gacc[ch].at[pl.ds(jc * CHUNK, CHUNK)],
                                    ctx.at[ch, pl.ds(chunk * CHUNK, CHUNK)],
                                )
                            return 0

                        lax.fori_loop(0, bcnt, wb_body, 0)


_gather_kernel = functools.partial(
    pl.kernel,
    out_type=tuple(
        jax.ShapeDtypeStruct((4, NPAD), jnp.float32) for _ in range(12)
    ),
    mesh=_mesh,
    compiler_params=_cparams,
    scratch_types=(
        [pltpu.VMEM_SHARED((4 * S,), jnp.float32)]
        + [pltpu.VMEM((CHUNK,), jnp.float32) for _ in range(3)]
        + [
            pltpu.VMEM((4096,), jnp.int32),
            pltpu.VMEM((4096,), jnp.float32),
            pltpu.VMEM((2048,), jnp.int32),
            pltpu.VMEM((2048,), jnp.float32),
            pltpu.VMEM((2048,), jnp.float32),
            pltpu.VMEM((4 * CHUNK,), jnp.float32),
            pltpu.VMEM((128,), jnp.int32),
        ]
        + [pltpu.VMEM((FIFO,), jnp.float32) for _ in range(3)]
        + [pltpu.VMEM((FIFO,), jnp.int32)]
        + [pltpu.VMEM((ACC,), jnp.float32) for _ in range(4)]
        + [pltpu.SemaphoreType.DMA]
    ),
)(_gather_body)


def kernel(xyz_for_creater, xyz_for_interp, feature):
    pad = NPAD - N
    cx = jnp.pad(xyz_for_creater[:, 0], (0, pad))
    cy = jnp.pad(xyz_for_creater[:, 1], (0, pad))
    cz = jnp.pad(xyz_for_creater[:, 2], (0, pad))
    f0 = jnp.pad(feature[:, 0], (0, pad))
    f1 = jnp.pad(feature[:, 1], (0, pad))
    f2 = jnp.pad(feature[:, 2], (0, pad))
    f3 = jnp.pad(feature[:, 3], (0, pad))
    ix = jnp.pad(xyz_for_interp[:, 0], (0, pad))
    iy = jnp.pad(xyz_for_interp[:, 1], (0, pad))
    iz = jnp.pad(xyz_for_interp[:, 2], (0, pad))
    grids = _scatter_kernel(cx, cy, cz, f0, f1, f2, f3)
    ctxs = _gather_kernel(ix, iy, iz, *grids)
    by_task = {ti: ctxs[k] for k, ti in enumerate(_KORDER)}
    cols = [by_task[t][:, :N].T for t in range(12)]
    return jnp.concatenate(cols, axis=1)
